# Initial kernel scaffold; baseline (speedup 1.0000x reference)
import jax
import jax.numpy as jnp
from jax.experimental import pallas as pl


def kernel(x, batch, down_w, down_gamma, down_beta, up_w, up_gamma, up_beta, gcn_w, gcn_b):
    raise NotImplementedError("write your pallas kernel here")


# speedup vs baseline: 572.3698x; 572.3698x over previous
import functools
import numpy as np
import jax
import jax.numpy as jnp
from jax.experimental import pallas as pl

_K = 4


@functools.lru_cache(maxsize=None)
def _gcn_constants(B, V, HW, N):
    deg = np.ones(N, np.float64)
    edges = []
    for t in range(V - 1):
        for k in range(_K):
            s, d = t * HW, (t + 1) * HW + k
            edges.append((s, d))
            edges.append((d, s))
    for (_, c) in edges:
        deg[c] += 1.0
    dis = 1.0 / np.sqrt(deg)
    M = np.zeros((V * _K, V * _K), np.float64)
    for (r, c) in edges:
        qr = (r // HW) * _K + (r % HW)
        qc = (c // HW) * _K + (c % HW)
        M[qr, qc] += dis[r] * dis[c]
    invdeg = np.tile(1.0 / deg, B)[None, :]
    return (np.asarray(invdeg, np.float32), np.asarray(M, np.float32))


def _conv_bn_body(x_ref, w_ref, g_ref, b_ref, o_ref, *, Bn, V, HW):
    X = x_ref[...]
    w = w_ref[...]
    f32 = jnp.float32
    Z1 = jnp.dot(w[1], X, preferred_element_type=f32)
    Z0 = jnp.dot(w[0], X, preferred_element_type=f32)
    Z2 = jnp.dot(w[2], X, preferred_element_type=f32)
    R = Z1.shape[0]
    Z0 = Z0.reshape(R, Bn, V, HW)
    Z2 = Z2.reshape(R, Bn, V, HW)
    zpad = jnp.zeros((R, Bn, 1, HW), f32)
    Y = (Z1.reshape(R, Bn, V, HW)
         + jnp.concatenate([zpad, Z0[:, :, :-1, :]], axis=2)
         + jnp.concatenate([Z2[:, :, 1:, :], zpad], axis=2))
    Yf = Y.reshape(R, Bn * V * HW)
    mean = jnp.mean(Yf, axis=1, keepdims=True)
    var = jnp.mean((Yf - mean) ** 2, axis=1, keepdims=True)
    o_ref[...] = (Yf - mean) / jnp.sqrt(var + 1e-5) * g_ref[...] + b_ref[...]


def _gcn_body(y_ref, wt_ref, b_ref, inv_ref, m_ref, o_ref, *, Bn, V, HW):
    Y = y_ref[...]
    Wb = wt_ref[...]
    XW = jnp.dot(Wb, Y, preferred_element_type=jnp.float32)
    out = XW * inv_ref[...]
    R = XW.shape[0]
    Xs = XW.reshape(R, Bn, V, HW)[:, :, :, :_K]
    Xs2 = Xs.reshape(R * Bn, V * _K)
    contrib = jnp.dot(Xs2, m_ref[...], preferred_element_type=jnp.float32)
    contrib = contrib.reshape(R, Bn, V, _K)
    zpad = jnp.zeros((R, Bn, V, HW - _K), jnp.float32)
    out = out + jnp.concatenate([contrib, zpad], axis=3).reshape(R, -1)
    o_ref[...] = out + b_ref[...]


def _run_conv_bn(x_cm, w3, gamma, beta, Bn, V, HW, grid_rows=2):
    C = x_cm.shape[0]
    NCOL = x_cm.shape[1]
    RB = C // grid_rows
    body = functools.partial(_conv_bn_body, Bn=Bn, V=V, HW=HW)
    return pl.pallas_call(
        body,
        grid=(grid_rows,),
        in_specs=[
            pl.BlockSpec((C, NCOL), lambda i: (0, 0)),
            pl.BlockSpec((3, RB, C), lambda i: (0, i, 0)),
            pl.BlockSpec((RB, 1), lambda i: (i, 0)),
            pl.BlockSpec((RB, 1), lambda i: (i, 0)),
        ],
        out_specs=pl.BlockSpec((RB, NCOL), lambda i: (i, 0)),
        out_shape=jax.ShapeDtypeStruct((C, NCOL), jnp.float32),
    )(x_cm, w3, gamma, beta)


def _run_gcn(y_cm, wt_t, bias, invdeg, M, Bn, V, HW, grid_rows=2):
    C = y_cm.shape[0]
    NCOL = y_cm.shape[1]
    RB = C // grid_rows
    VK = V * _K
    body = functools.partial(_gcn_body, Bn=Bn, V=V, HW=HW)
    return pl.pallas_call(
        body,
        grid=(grid_rows,),
        in_specs=[
            pl.BlockSpec((C, NCOL), lambda i: (0, 0)),
            pl.BlockSpec((RB, C), lambda i: (i, 0)),
            pl.BlockSpec((RB, 1), lambda i: (i, 0)),
            pl.BlockSpec((1, NCOL), lambda i: (0, 0)),
            pl.BlockSpec((VK, VK), lambda i: (0, 0)),
        ],
        out_specs=pl.BlockSpec((RB, NCOL), lambda i: (i, 0)),
        out_shape=jax.ShapeDtypeStruct((C, NCOL), jnp.float32),
    )(y_cm, wt_t, bias, invdeg, M)


def kernel(x, batch, down_w, down_gamma, down_beta, up_w, up_gamma, up_beta,
           gcn_w, gcn_b):
    tlen, C, H, W = x.shape
    try:
        Bn = int(batch)
    except Exception:
        Bn = 4
    V = tlen // Bn
    HW = H * W
    NCOL = Bn * V * HW
    N = V * HW

    invdeg_np, M_np = _gcn_constants(Bn, V, HW, N)
    invdeg = jnp.asarray(invdeg_np)
    M = jnp.asarray(M_np)

    x_cm = jnp.transpose(x.reshape(tlen, C, HW), (1, 0, 2)).reshape(C, NCOL)

    dw3 = jnp.transpose(down_w.reshape(C, C, 3), (2, 0, 1))
    uw3 = jnp.transpose(up_w.reshape(C, C, 3), (2, 0, 1))

    y = _run_conv_bn(x_cm, dw3, down_gamma.reshape(C, 1),
                     down_beta.reshape(C, 1), Bn, V, HW)
    g = _run_gcn(y, jnp.transpose(gcn_w), gcn_b.reshape(C, 1),
                 invdeg, M, Bn, V, HW)
    z = _run_conv_bn(g, uw3, up_gamma.reshape(C, 1),
                     up_beta.reshape(C, 1), Bn, V, HW)

    out = jnp.transpose(z.reshape(C, Bn, V, HW), (1, 2, 0, 3))
    return out.reshape(tlen, C, H, W)
